# Initial kernel scaffold; baseline (speedup 1.0000x reference)
import jax
import jax.numpy as jnp
from jax.experimental import pallas as pl


def kernel(x, edge_index, edge_attr, goal_feat, batch, Wf1, bf1, Ws1, bs1, g1, be1, Wf2, bf2, Ws2, bs2, g2, be2, W3, b3, W4, b4, Wd1, bd1, Wd2, bd2):
    raise NotImplementedError("write your pallas kernel here")


# speedup vs baseline: 1.1181x; 1.1181x over previous
import functools
import math

import jax
import jax.numpy as jnp
from jax import lax
from jax.experimental import pallas as pl

N = 10000
E = 320000
C = 128
D = 16
H = 128

_BN_SCALE = 1.0 / math.sqrt(1.0 + 1e-5)


def _mm_body(x_ref, w_ref, b_ref, o_ref, *, act):
    acc = jnp.dot(x_ref[...], w_ref[...], preferred_element_type=jnp.float32)
    acc = acc + b_ref[...][None, :]
    if act == "relu":
        acc = jnp.maximum(acc, 0.0)
    o_ref[...] = acc


def _mm(x, w, b=None, act="none", bm=2000):
    m, k = x.shape
    n = w.shape[1]
    assert m % bm == 0, (m, bm)
    if b is None:
        b = jnp.zeros((n,), jnp.float32)
    return pl.pallas_call(
        functools.partial(_mm_body, act=act),
        grid=(m // bm,),
        in_specs=[
            pl.BlockSpec((bm, k), lambda i: (i, 0)),
            pl.BlockSpec((k, n), lambda i: (0, 0)),
            pl.BlockSpec((n,), lambda i: (0,)),
        ],
        out_specs=pl.BlockSpec((bm, n), lambda i: (i, 0)),
        out_shape=jax.ShapeDtypeStruct((m, n), jnp.float32),
    )(x, w, b)


def _gate_body(pg_ref, qg_ref, et_ref, o_ref):
    z = pg_ref[...] + qg_ref[...] + et_ref[...]
    zf = z[:, :C]
    zs = z[:, C:]
    o_ref[...] = jax.nn.sigmoid(zf) * jax.nn.softplus(zs)


def _gate(pg, qg, et, bm=2000):
    m = pg.shape[0]
    return pl.pallas_call(
        _gate_body,
        grid=(m // bm,),
        in_specs=[pl.BlockSpec((bm, 2 * C), lambda i: (i, 0))] * 3,
        out_specs=pl.BlockSpec((bm, C), lambda i: (i, 0)),
        out_shape=jax.ShapeDtypeStruct((m, C), jnp.float32),
    )(pg, qg, et)


def _cg_epilogue_body(agg_ref, x_ref, g_ref, be_ref, o_ref):
    agg = agg_ref[...] * _BN_SCALE * g_ref[...][None, :] + be_ref[...][None, :]
    o_ref[...] = jnp.maximum(agg + x_ref[...], 0.0)


def _cg_epilogue(agg, x, g, be, bm=2000):
    m = agg.shape[0]
    return pl.pallas_call(
        _cg_epilogue_body,
        grid=(m // bm,),
        in_specs=[
            pl.BlockSpec((bm, C), lambda i: (i, 0)),
            pl.BlockSpec((bm, C), lambda i: (i, 0)),
            pl.BlockSpec((C,), lambda i: (0,)),
            pl.BlockSpec((C,), lambda i: (0,)),
        ],
        out_specs=pl.BlockSpec((bm, C), lambda i: (i, 0)),
        out_shape=jax.ShapeDtypeStruct((m, C), jnp.float32),
    )(agg, x, g, be)


def _gcn_epilogue_body(agg_ref, xw_ref, inv_deg_ref, b_ref, o_ref):
    out = agg_ref[...] + xw_ref[...] * inv_deg_ref[...] + b_ref[...][None, :]
    o_ref[...] = jnp.maximum(out, 0.0)


def _gcn_epilogue(agg, xw, inv_deg, b, bm=2000):
    m = agg.shape[0]
    return pl.pallas_call(
        _gcn_epilogue_body,
        grid=(m // bm,),
        in_specs=[
            pl.BlockSpec((bm, C), lambda i: (i, 0)),
            pl.BlockSpec((bm, C), lambda i: (i, 0)),
            pl.BlockSpec((bm, 1), lambda i: (i, 0)),
            pl.BlockSpec((C,), lambda i: (0,)),
        ],
        out_specs=pl.BlockSpec((bm, C), lambda i: (i, 0)),
        out_shape=jax.ShapeDtypeStruct((m, C), jnp.float32),
    )(agg, xw, inv_deg, b)


def _cgconv(h, src, dst, et, Wp, Wq, g, be):
    p = _mm(h, Wp)
    q = _mm(h, Wq)
    m = _gate(p[dst], q[src], et)
    agg = jnp.zeros((N, C), jnp.float32).at[dst].add(m)
    return _cg_epilogue(agg, h, g, be)


def _gcnconv(h, src, dst, dis, inv_deg, W, b):
    xw = _mm(h, W)
    norm = dis[src] * dis[dst]
    agg = jnp.zeros((N, C), jnp.float32).at[dst].add(xw[src] * norm[:, None])
    return _gcn_epilogue(agg, xw, inv_deg, b)


def kernel(x, edge_index, edge_attr, goal_feat, batch, Wf1, bf1, Ws1, bs1, g1, be1, Wf2, bf2, Ws2, bs2, g2, be2, W3, b3, W4, b4, Wd1, bd1, Wd2, bd2):
    src, dst = edge_index[0], edge_index[1]

    Wp1 = jnp.concatenate([Wf1[:C], Ws1[:C]], axis=1)
    Wq1 = jnp.concatenate([Wf1[C:2 * C], Ws1[C:2 * C]], axis=1)
    Wet1 = jnp.concatenate([Wf1[2 * C:], Ws1[2 * C:]], axis=1)
    bet1 = jnp.concatenate([bf1, bs1])
    Wp2 = jnp.concatenate([Wf2[:C], Ws2[:C]], axis=1)
    Wq2 = jnp.concatenate([Wf2[C:2 * C], Ws2[C:2 * C]], axis=1)
    Wet2 = jnp.concatenate([Wf2[2 * C:], Ws2[2 * C:]], axis=1)
    bet2 = jnp.concatenate([bf2, bs2])

    et1 = _mm(edge_attr, Wet1, bet1)
    et2 = _mm(edge_attr, Wet2, bet2)

    h = _cgconv(x, src, dst, et1, Wp1, Wq1, g1, be1)
    h = _cgconv(h, src, dst, et2, Wp2, Wq2, g2, be2)

    deg = jnp.zeros((N,), jnp.float32).at[dst].add(1.0) + 1.0
    dis = lax.rsqrt(deg)
    inv_deg = (1.0 / deg)[:, None]

    h = _gcnconv(h, src, dst, dis, inv_deg, W3, b3)
    h = _gcnconv(h, src, dst, dis, inv_deg, W4, b4)

    cat = jnp.concatenate([h, goal_feat], axis=1)
    t = _mm(cat, Wd1, bd1, act="relu")
    Wd2p = jnp.concatenate([Wd2, jnp.zeros((C, 127), jnp.float32)], axis=1)
    bd2p = jnp.concatenate([bd2, jnp.zeros((127,), jnp.float32)])
    pred = _mm(t, Wd2p, bd2p)[:, :1]
    return (pred, h)


# speedup vs baseline: 1.4282x; 1.2774x over previous
import functools
import math

import jax
import jax.numpy as jnp
from jax import lax
from jax.experimental import pallas as pl
from jax.experimental.pallas import tpu as pltpu
from jax.experimental.pallas import tpu_sc as plsc

N = 10000
E = 320000
C = 128
D = 16
H = 128

NC = 2
NS = 16
NW = NC * NS
NPAD = 10240
RPT = NPAD // NS
EPW = E // NW
CG_CH = 40
CG_NCHUNK = EPW // CG_CH
GCN_CH = 80
GCN_NCHUNK = EPW // GCN_CH

_BN_SCALE = 1.0 / math.sqrt(1.0 + 1e-5)

_MESH = plsc.VectorSubcoreMesh(core_axis_name="c", subcore_axis_name="s")


def _mm_body(x_ref, w_ref, b_ref, rs_ref, o_ref, *, act):
    acc = jnp.dot(x_ref[...], w_ref[...], preferred_element_type=jnp.float32)
    acc = acc + b_ref[...][None, :]
    if act == "relu":
        acc = jnp.maximum(acc, 0.0)
    o_ref[...] = acc * rs_ref[...]


def _mm(x, w, b=None, act="none", rs=None, bm=2000):
    m, k = x.shape
    n = w.shape[1]
    assert m % bm == 0, (m, bm)
    if b is None:
        b = jnp.zeros((n,), jnp.float32)
    if rs is None:
        rs = jnp.ones((m, 1), jnp.float32)
    return pl.pallas_call(
        functools.partial(_mm_body, act=act),
        grid=(m // bm,),
        in_specs=[
            pl.BlockSpec((bm, k), lambda i: (i, 0)),
            pl.BlockSpec((k, n), lambda i: (0, 0)),
            pl.BlockSpec((n,), lambda i: (0,)),
            pl.BlockSpec((bm, 1), lambda i: (i, 0)),
        ],
        out_specs=pl.BlockSpec((bm, n), lambda i: (i, 0)),
        out_shape=jax.ShapeDtypeStruct((m, n), jnp.float32),
    )(x, w, b, rs)


def _mm2_body(x_ref, y_ref, wx_ref, wy_ref, b_ref, o_ref, *, act):
    acc = jnp.dot(x_ref[...], wx_ref[...], preferred_element_type=jnp.float32)
    acc = acc + jnp.dot(y_ref[...], wy_ref[...], preferred_element_type=jnp.float32)
    acc = acc + b_ref[...][None, :]
    if act == "relu":
        acc = jnp.maximum(acc, 0.0)
    o_ref[...] = acc


def _mm2(x, y, wx, wy, b, act="none", bm=2000):
    m, kx = x.shape
    ky = y.shape[1]
    n = wx.shape[1]
    return pl.pallas_call(
        functools.partial(_mm2_body, act=act),
        grid=(m // bm,),
        in_specs=[
            pl.BlockSpec((bm, kx), lambda i: (i, 0)),
            pl.BlockSpec((bm, ky), lambda i: (i, 0)),
            pl.BlockSpec((kx, n), lambda i: (0, 0)),
            pl.BlockSpec((ky, n), lambda i: (0, 0)),
            pl.BlockSpec((n,), lambda i: (0,)),
        ],
        out_specs=pl.BlockSpec((bm, n), lambda i: (i, 0)),
        out_shape=jax.ShapeDtypeStruct((m, n), jnp.float32),
    )(x, y, wx, wy, b)


def _cg_epilogue_body(a0_ref, a1_ref, x_ref, g_ref, be_ref, o_ref):
    agg = a0_ref[...] + a1_ref[...]
    agg = agg * _BN_SCALE * g_ref[...][None, :] + be_ref[...][None, :]
    o_ref[...] = jnp.maximum(agg + x_ref[...], 0.0)


def _cg_epilogue(a0, a1, x, g, be, bm=2000):
    m = x.shape[0]
    return pl.pallas_call(
        _cg_epilogue_body,
        grid=(m // bm,),
        in_specs=[
            pl.BlockSpec((bm, C), lambda i: (i, 0)),
            pl.BlockSpec((bm, C), lambda i: (i, 0)),
            pl.BlockSpec((bm, C), lambda i: (i, 0)),
            pl.BlockSpec((C,), lambda i: (0,)),
            pl.BlockSpec((C,), lambda i: (0,)),
        ],
        out_specs=pl.BlockSpec((bm, C), lambda i: (i, 0)),
        out_shape=jax.ShapeDtypeStruct((m, C), jnp.float32),
    )(a0, a1, x, g, be)


def _gcn_epilogue_body(a0_ref, a1_ref, xs_ref, dis_ref, b_ref, o_ref):
    out = (a0_ref[...] + a1_ref[...] + xs_ref[...]) * dis_ref[...] + b_ref[...][None, :]
    o_ref[...] = jnp.maximum(out, 0.0)


def _gcn_epilogue(a0, a1, xs, dis_n, b, bm=2000):
    m = xs.shape[0]
    return pl.pallas_call(
        _gcn_epilogue_body,
        grid=(m // bm,),
        in_specs=[
            pl.BlockSpec((bm, C), lambda i: (i, 0)),
            pl.BlockSpec((bm, C), lambda i: (i, 0)),
            pl.BlockSpec((bm, C), lambda i: (i, 0)),
            pl.BlockSpec((bm, 1), lambda i: (i, 0)),
            pl.BlockSpec((C,), lambda i: (0,)),
        ],
        out_specs=pl.BlockSpec((bm, C), lambda i: (i, 0)),
        out_shape=jax.ShapeDtypeStruct((m, C), jnp.float32),
    )(a0, a1, xs, dis_n, b)


def _deg_finish_body(d_ref, dis_ref):
    deg = d_ref[0] + d_ref[1] + 1.0
    dis_ref[...] = lax.rsqrt(deg)


def _deg_finish(deg_raw):
    d = deg_raw.reshape(2, NPAD // 128, 128)
    dis = pl.pallas_call(
        _deg_finish_body,
        grid=(1,),
        in_specs=[pl.BlockSpec((2, NPAD // 128, 128), lambda i: (0, 0, 0))],
        out_specs=pl.BlockSpec((NPAD // 128, 128), lambda i: (0, 0)),
        out_shape=jax.ShapeDtypeStruct((NPAD // 128, 128), jnp.float32),
    )(d)
    return dis.reshape(NPAD)


def _sigmoid16(x):
    return 1.0 / (1.0 + jnp.exp(-x))


def _softplus16(x):
    u = jnp.exp(-jnp.abs(x))
    t = u / (2.0 + u)
    t2 = t * t
    p = t * (1.0 + t2 * (1.0 / 3.0 + t2 * (0.2 + t2 * (1.0 / 7.0))))
    return jnp.maximum(x, 0.0) + 2.0 * p


_Z16 = lambda: jnp.zeros((16,), jnp.float32)


def _sc_cg_body(p_hbm, q_hbm, et_hbm, dst_hbm, src_hbm,
                out_hbm, deg_hbm,
                dstv, srcv, pbuf, qbuf, etbuf, mbuf, onesv, zbuf, z1buf,
                acc, acc1, sem_p, sem_q, sem_e):
    cid = lax.axis_index("c")
    sid = lax.axis_index("s")
    w = sid * NC + cid
    z16 = _Z16()

    def zrow(i, carry):
        for r in range(8):
            zbuf[i, pl.ds(r * 16, 16)] = z16
        return carry
    lax.fori_loop(0, 16, zrow, 0)
    z1buf[...] = z16

    ones16 = jnp.ones((16,), jnp.float32)

    onesv[pl.ds(0, 16)] = ones16
    onesv[pl.ds(16, 16)] = ones16
    onesv[pl.ds(CG_CH - 16, 16)] = ones16

    def zacc(b, carry):
        pltpu.sync_copy(zbuf, acc.at[pl.ds(sid * RPT + b * 16, 16)])
        pltpu.sync_copy(z1buf, acc1.at[pl.ds(sid * RPT + b * 16, 16)])
        return carry
    lax.fori_loop(0, RPT // 16, zacc, 0)
    plsc.subcore_barrier()

    base0 = w * EPW

    def chunk(c, carry):
        base = base0 + c * CG_CH
        pltpu.sync_copy(dst_hbm.at[pl.ds(base, CG_CH)], dstv)
        pltpu.sync_copy(src_hbm.at[pl.ds(base, CG_CH)], srcv)
        cp_p = pltpu.async_copy(p_hbm.at[dstv], pbuf, sem_p)
        cp_q = pltpu.async_copy(q_hbm.at[srcv], qbuf, sem_q)
        cp_e = pltpu.async_copy(et_hbm.at[pl.ds(base, CG_CH)], etbuf, sem_e)
        pltpu.sync_copy(onesv, acc1.at[dstv], add=True)
        cp_p.wait()
        cp_q.wait()
        cp_e.wait()

        def edge(i, carry2):
            for r in range(8):
                lo = pl.ds(r * 16, 16)
                hi = pl.ds(128 + r * 16, 16)
                zf = pbuf[i, lo] + qbuf[i, lo] + etbuf[i, lo]
                zs = pbuf[i, hi] + qbuf[i, hi] + etbuf[i, hi]
                mbuf[i, lo] = _sigmoid16(zf) * _softplus16(zs)
            return carry2
        lax.fori_loop(0, CG_CH, edge, 0)
        pltpu.sync_copy(mbuf, acc.at[dstv], add=True)
        return carry
    lax.fori_loop(0, CG_NCHUNK, chunk, 0)
    plsc.subcore_barrier()

    pltpu.sync_copy(acc.at[pl.ds(sid * RPT, RPT)],
                    out_hbm.at[cid, pl.ds(sid * RPT, RPT)])
    pltpu.sync_copy(acc1.at[pl.ds(sid * RPT, RPT)],
                    deg_hbm.at[cid, pl.ds(sid * RPT, RPT)])


_sc_cg = pl.kernel(
    _sc_cg_body,
    out_type=[
        jax.ShapeDtypeStruct((NC, NPAD, C), jnp.float32),
        jax.ShapeDtypeStruct((NC, NPAD), jnp.float32),
    ],
    mesh=_MESH,
    scratch_types=[
        pltpu.VMEM((CG_CH,), jnp.int32),
        pltpu.VMEM((CG_CH,), jnp.int32),
        pltpu.VMEM((CG_CH, 2 * C), jnp.float32),
        pltpu.VMEM((CG_CH, 2 * C), jnp.float32),
        pltpu.VMEM((CG_CH, 2 * C), jnp.float32),
        pltpu.VMEM((CG_CH, C), jnp.float32),
        pltpu.VMEM((CG_CH,), jnp.float32),
        pltpu.VMEM((16, C), jnp.float32),
        pltpu.VMEM((16,), jnp.float32),
        pltpu.VMEM_SHARED((NPAD, C), jnp.float32),
        pltpu.VMEM_SHARED((NPAD,), jnp.float32),
        pltpu.SemaphoreType.DMA,
        pltpu.SemaphoreType.DMA,
        pltpu.SemaphoreType.DMA,
    ],
)


def _sc_gcn_body(xs_hbm, dst_hbm, src_hbm,
                 out_hbm,
                 dstv, srcv, rbuf, zbuf,
                 acc, sem_r):
    cid = lax.axis_index("c")
    sid = lax.axis_index("s")
    w = sid * NC + cid
    z16 = _Z16()

    def zrow(i, carry):
        for r in range(8):
            zbuf[i, pl.ds(r * 16, 16)] = z16
        return carry
    lax.fori_loop(0, 16, zrow, 0)

    def zacc(b, carry):
        pltpu.sync_copy(zbuf, acc.at[pl.ds(sid * RPT + b * 16, 16)])
        return carry
    lax.fori_loop(0, RPT // 16, zacc, 0)
    plsc.subcore_barrier()

    base0 = w * EPW

    def chunk(c, carry):
        base = base0 + c * GCN_CH
        pltpu.sync_copy(dst_hbm.at[pl.ds(base, GCN_CH)], dstv)
        pltpu.sync_copy(src_hbm.at[pl.ds(base, GCN_CH)], srcv)
        pltpu.async_copy(xs_hbm.at[srcv], rbuf, sem_r).wait()
        pltpu.sync_copy(rbuf, acc.at[dstv], add=True)
        return carry
    lax.fori_loop(0, GCN_NCHUNK, chunk, 0)
    plsc.subcore_barrier()

    pltpu.sync_copy(acc.at[pl.ds(sid * RPT, RPT)],
                    out_hbm.at[cid, pl.ds(sid * RPT, RPT)])


_sc_gcn = pl.kernel(
    _sc_gcn_body,
    out_type=jax.ShapeDtypeStruct((NC, NPAD, C), jnp.float32),
    mesh=_MESH,
    scratch_types=[
        pltpu.VMEM((GCN_CH,), jnp.int32),
        pltpu.VMEM((GCN_CH,), jnp.int32),
        pltpu.VMEM((GCN_CH, C), jnp.float32),
        pltpu.VMEM((16, C), jnp.float32),
        pltpu.VMEM_SHARED((NPAD, C), jnp.float32),
        pltpu.SemaphoreType.DMA,
    ],
)


def _cgconv(h, src, dst, et, Wp, Wq, g, be):
    p = _mm(h, Wp)
    q = _mm(h, Wq)
    agg, deg_raw = _sc_cg(p, q, et, dst, src)
    return _cg_epilogue(agg[0, :N], agg[1, :N], h, g, be), deg_raw


def _gcnconv(h, src, dst, dis_n, W, b):
    xs = _mm(h, W, rs=dis_n)
    agg = _sc_gcn(xs, dst, src)
    return _gcn_epilogue(agg[0, :N], agg[1, :N], xs, dis_n, b)


def kernel(x, edge_index, edge_attr, goal_feat, batch, Wf1, bf1, Ws1, bs1, g1, be1, Wf2, bf2, Ws2, bs2, g2, be2, W3, b3, W4, b4, Wd1, bd1, Wd2, bd2):
    src, dst = edge_index[0], edge_index[1]

    Wp1 = jnp.concatenate([Wf1[:C], Ws1[:C]], axis=1)
    Wq1 = jnp.concatenate([Wf1[C:2 * C], Ws1[C:2 * C]], axis=1)
    Wet1 = jnp.concatenate([Wf1[2 * C:], Ws1[2 * C:]], axis=1)
    bet1 = jnp.concatenate([bf1, bs1])
    Wp2 = jnp.concatenate([Wf2[:C], Ws2[:C]], axis=1)
    Wq2 = jnp.concatenate([Wf2[C:2 * C], Ws2[C:2 * C]], axis=1)
    Wet2 = jnp.concatenate([Wf2[2 * C:], Ws2[2 * C:]], axis=1)
    bet2 = jnp.concatenate([bf2, bs2])

    et1 = _mm(edge_attr, Wet1, bet1)
    et2 = _mm(edge_attr, Wet2, bet2)

    h, deg_raw = _cgconv(x, src, dst, et1, Wp1, Wq1, g1, be1)
    h, _ = _cgconv(h, src, dst, et2, Wp2, Wq2, g2, be2)

    dis_n = _deg_finish(deg_raw)[:N, None]

    h = _gcnconv(h, src, dst, dis_n, W3, b3)
    h = _gcnconv(h, src, dst, dis_n, W4, b4)

    t = _mm2(h, goal_feat, Wd1[:C], Wd1[C:], bd1, act="relu")
    Wd2p = jnp.concatenate([Wd2, jnp.zeros((C, 127), jnp.float32)], axis=1)
    bd2p = jnp.concatenate([bd2, jnp.zeros((127,), jnp.float32)])
    pred = _mm(t, Wd2p, bd2p)[:, :1]
    return (pred, h)


# speedup vs baseline: 1.5335x; 1.0737x over previous
import functools
import math

import jax
import jax.numpy as jnp
from jax import lax
from jax.experimental import pallas as pl
from jax.experimental.pallas import tpu as pltpu
from jax.experimental.pallas import tpu_sc as plsc

N = 10000
E = 320000
C = 128
D = 16
H = 128

NC = 2
NS = 16
NW = NC * NS
NPAD = 10240
RPT = NPAD // NS
EPW = E // NW
CG_CH = 40
CG_NCHUNK = EPW // CG_CH
GCN_CH = 80
GCN_NCHUNK = EPW // GCN_CH

_BN_SCALE = 1.0 / math.sqrt(1.0 + 1e-5)

_MESH = plsc.VectorSubcoreMesh(core_axis_name="c", subcore_axis_name="s")


def _mm_body(x_ref, w_ref, b_ref, rs_ref, o_ref, *, act):
    acc = jnp.dot(x_ref[...], w_ref[...], preferred_element_type=jnp.float32)
    acc = acc + b_ref[...][None, :]
    if act == "relu":
        acc = jnp.maximum(acc, 0.0)
    o_ref[...] = acc * rs_ref[...]


def _mm(x, w, b=None, act="none", rs=None, bm=2000):
    m, k = x.shape
    n = w.shape[1]
    assert m % bm == 0, (m, bm)
    if b is None:
        b = jnp.zeros((n,), jnp.float32)
    if rs is None:
        rs = jnp.ones((m, 1), jnp.float32)
    return pl.pallas_call(
        functools.partial(_mm_body, act=act),
        grid=(m // bm,),
        in_specs=[
            pl.BlockSpec((bm, k), lambda i: (i, 0)),
            pl.BlockSpec((k, n), lambda i: (0, 0)),
            pl.BlockSpec((n,), lambda i: (0,)),
            pl.BlockSpec((bm, 1), lambda i: (i, 0)),
        ],
        out_specs=pl.BlockSpec((bm, n), lambda i: (i, 0)),
        out_shape=jax.ShapeDtypeStruct((m, n), jnp.float32),
    )(x, w, b, rs)


def _mm2_body(x_ref, y_ref, wx_ref, wy_ref, b_ref, o_ref, *, act):
    acc = jnp.dot(x_ref[...], wx_ref[...], preferred_element_type=jnp.float32)
    acc = acc + jnp.dot(y_ref[...], wy_ref[...], preferred_element_type=jnp.float32)
    acc = acc + b_ref[...][None, :]
    if act == "relu":
        acc = jnp.maximum(acc, 0.0)
    o_ref[...] = acc


def _mm2(x, y, wx, wy, b, act="none", bm=2000):
    m, kx = x.shape
    ky = y.shape[1]
    n = wx.shape[1]
    return pl.pallas_call(
        functools.partial(_mm2_body, act=act),
        grid=(m // bm,),
        in_specs=[
            pl.BlockSpec((bm, kx), lambda i: (i, 0)),
            pl.BlockSpec((bm, ky), lambda i: (i, 0)),
            pl.BlockSpec((kx, n), lambda i: (0, 0)),
            pl.BlockSpec((ky, n), lambda i: (0, 0)),
            pl.BlockSpec((n,), lambda i: (0,)),
        ],
        out_specs=pl.BlockSpec((bm, n), lambda i: (i, 0)),
        out_shape=jax.ShapeDtypeStruct((m, n), jnp.float32),
    )(x, y, wx, wy, b)


def _cg_epilogue_body(a0_ref, a1_ref, x_ref, g_ref, be_ref, o_ref):
    agg = a0_ref[...] + a1_ref[...]
    agg = agg * _BN_SCALE * g_ref[...][None, :] + be_ref[...][None, :]
    o_ref[...] = jnp.maximum(agg + x_ref[...], 0.0)


def _cg_epilogue(a0, a1, x, g, be, bm=2000):
    m = x.shape[0]
    return pl.pallas_call(
        _cg_epilogue_body,
        grid=(m // bm,),
        in_specs=[
            pl.BlockSpec((bm, C), lambda i: (i, 0)),
            pl.BlockSpec((bm, C), lambda i: (i, 0)),
            pl.BlockSpec((bm, C), lambda i: (i, 0)),
            pl.BlockSpec((C,), lambda i: (0,)),
            pl.BlockSpec((C,), lambda i: (0,)),
        ],
        out_specs=pl.BlockSpec((bm, C), lambda i: (i, 0)),
        out_shape=jax.ShapeDtypeStruct((m, C), jnp.float32),
    )(a0, a1, x, g, be)


def _gcn_epilogue_body(a0_ref, a1_ref, xs_ref, dis_ref, b_ref, o_ref):
    out = (a0_ref[...] + a1_ref[...] + xs_ref[...]) * dis_ref[...] + b_ref[...][None, :]
    o_ref[...] = jnp.maximum(out, 0.0)


def _gcn_epilogue(a0, a1, xs, dis_n, b, bm=2000):
    m = xs.shape[0]
    return pl.pallas_call(
        _gcn_epilogue_body,
        grid=(m // bm,),
        in_specs=[
            pl.BlockSpec((bm, C), lambda i: (i, 0)),
            pl.BlockSpec((bm, C), lambda i: (i, 0)),
            pl.BlockSpec((bm, C), lambda i: (i, 0)),
            pl.BlockSpec((bm, 1), lambda i: (i, 0)),
            pl.BlockSpec((C,), lambda i: (0,)),
        ],
        out_specs=pl.BlockSpec((bm, C), lambda i: (i, 0)),
        out_shape=jax.ShapeDtypeStruct((m, C), jnp.float32),
    )(a0, a1, xs, dis_n, b)


def _deg_finish_body(d_ref, dis_ref):
    deg = d_ref[0] + d_ref[1] + 1.0
    dis_ref[...] = lax.rsqrt(deg)


def _deg_finish(deg_raw):
    d = deg_raw.reshape(2, NPAD // 128, 128)
    dis = pl.pallas_call(
        _deg_finish_body,
        grid=(1,),
        in_specs=[pl.BlockSpec((2, NPAD // 128, 128), lambda i: (0, 0, 0))],
        out_specs=pl.BlockSpec((NPAD // 128, 128), lambda i: (0, 0)),
        out_shape=jax.ShapeDtypeStruct((NPAD // 128, 128), jnp.float32),
    )(d)
    return dis.reshape(NPAD)


_RC = (0.9999998947750498, -0.9999878238439907, 0.9996518341404294,
       -0.9956737601052322, 0.9708136591421862, -0.8797876642756245,
       0.6745504013258026, -0.3861259763572058, 0.14007357551854913,
       -0.023514213554301073)
_LC = (9.016290541952188e-08, 0.9999914792344704, -0.49980144961105805,
       0.3313355433015402, -0.23919512619745695, 0.16479062872968117,
       -0.09232023232625802, 0.034421614309399946, -0.006075432040828808)


def _sigmoid16(x):
    v = jnp.exp(-jnp.abs(x))
    r = _RC[9]
    for c in _RC[8::-1]:
        r = r * v + c
    return jnp.where(x < 0.0, 1.0 - r, r)


def _softplus16(x):
    v = jnp.exp(-jnp.abs(x))
    l = _LC[8]
    for c in _LC[7::-1]:
        l = l * v + c
    return jnp.maximum(x, 0.0) + l


_Z16 = lambda: jnp.zeros((16,), jnp.float32)


def _sc_cg_body(p_hbm, q_hbm, et_hbm, dst_hbm, src_hbm,
                out_hbm, deg_hbm,
                dstv, srcv, pbuf, qbuf, etbuf, mbuf, onesv, zbuf, z1buf,
                acc, acc1, sem_p, sem_q, sem_e):
    cid = lax.axis_index("c")
    sid = lax.axis_index("s")
    w = sid * NC + cid
    z16 = _Z16()

    def zrow(i, carry):
        for r in range(8):
            zbuf[i, pl.ds(r * 16, 16)] = z16
        return carry
    lax.fori_loop(0, 16, zrow, 0)
    z1buf[...] = z16

    ones16 = jnp.ones((16,), jnp.float32)

    onesv[pl.ds(0, 16)] = ones16
    onesv[pl.ds(16, 16)] = ones16
    onesv[pl.ds(CG_CH - 16, 16)] = ones16

    def zacc(b, carry):
        pltpu.sync_copy(zbuf, acc.at[pl.ds(sid * RPT + b * 16, 16)])
        pltpu.sync_copy(z1buf, acc1.at[pl.ds(sid * RPT + b * 16, 16)])
        return carry
    lax.fori_loop(0, RPT // 16, zacc, 0)
    plsc.subcore_barrier()

    base0 = w * EPW

    def chunk(c, carry):
        base = base0 + c * CG_CH
        pltpu.sync_copy(dst_hbm.at[pl.ds(base, CG_CH)], dstv)
        pltpu.sync_copy(src_hbm.at[pl.ds(base, CG_CH)], srcv)
        cp_p = pltpu.async_copy(p_hbm.at[dstv], pbuf, sem_p)
        cp_q = pltpu.async_copy(q_hbm.at[srcv], qbuf, sem_q)
        cp_e = pltpu.async_copy(et_hbm.at[pl.ds(base, CG_CH)], etbuf, sem_e)
        pltpu.sync_copy(onesv, acc1.at[dstv], add=True)
        cp_p.wait()
        cp_q.wait()
        cp_e.wait()

        def edge(i, carry2):
            for r in range(8):
                lo = pl.ds(r * 16, 16)
                hi = pl.ds(128 + r * 16, 16)
                zf = pbuf[i, lo] + qbuf[i, lo] + etbuf[i, lo]
                zs = pbuf[i, hi] + qbuf[i, hi] + etbuf[i, hi]
                mbuf[i, lo] = _sigmoid16(zf) * _softplus16(zs)
            return carry2
        lax.fori_loop(0, CG_CH, edge, 0, unroll=2)
        pltpu.sync_copy(mbuf, acc.at[dstv], add=True)
        return carry
    lax.fori_loop(0, CG_NCHUNK, chunk, 0)
    plsc.subcore_barrier()

    pltpu.sync_copy(acc.at[pl.ds(sid * RPT, RPT)],
                    out_hbm.at[cid, pl.ds(sid * RPT, RPT)])
    pltpu.sync_copy(acc1.at[pl.ds(sid * RPT, RPT)],
                    deg_hbm.at[cid, pl.ds(sid * RPT, RPT)])


_sc_cg = pl.kernel(
    _sc_cg_body,
    out_type=[
        jax.ShapeDtypeStruct((NC, NPAD, C), jnp.float32),
        jax.ShapeDtypeStruct((NC, NPAD), jnp.float32),
    ],
    mesh=_MESH,
    scratch_types=[
        pltpu.VMEM((CG_CH,), jnp.int32),
        pltpu.VMEM((CG_CH,), jnp.int32),
        pltpu.VMEM((CG_CH, 2 * C), jnp.float32),
        pltpu.VMEM((CG_CH, 2 * C), jnp.float32),
        pltpu.VMEM((CG_CH, 2 * C), jnp.float32),
        pltpu.VMEM((CG_CH, C), jnp.float32),
        pltpu.VMEM((CG_CH,), jnp.float32),
        pltpu.VMEM((16, C), jnp.float32),
        pltpu.VMEM((16,), jnp.float32),
        pltpu.VMEM_SHARED((NPAD, C), jnp.float32),
        pltpu.VMEM_SHARED((NPAD,), jnp.float32),
        pltpu.SemaphoreType.DMA,
        pltpu.SemaphoreType.DMA,
        pltpu.SemaphoreType.DMA,
    ],
)


def _sc_gcn_body(xs_hbm, dst_hbm, src_hbm,
                 out_hbm,
                 dstv, srcv, rbuf, zbuf,
                 acc, sem_r):
    cid = lax.axis_index("c")
    sid = lax.axis_index("s")
    w = sid * NC + cid
    z16 = _Z16()

    def zrow(i, carry):
        for r in range(8):
            zbuf[i, pl.ds(r * 16, 16)] = z16
        return carry
    lax.fori_loop(0, 16, zrow, 0)

    def zacc(b, carry):
        pltpu.sync_copy(zbuf, acc.at[pl.ds(sid * RPT + b * 16, 16)])
        return carry
    lax.fori_loop(0, RPT // 16, zacc, 0)
    plsc.subcore_barrier()

    base0 = w * EPW

    def chunk(c, carry):
        base = base0 + c * GCN_CH
        pltpu.sync_copy(dst_hbm.at[pl.ds(base, GCN_CH)], dstv)
        pltpu.sync_copy(src_hbm.at[pl.ds(base, GCN_CH)], srcv)
        pltpu.async_copy(xs_hbm.at[srcv], rbuf, sem_r).wait()
        pltpu.sync_copy(rbuf, acc.at[dstv], add=True)
        return carry
    lax.fori_loop(0, GCN_NCHUNK, chunk, 0)
    plsc.subcore_barrier()

    pltpu.sync_copy(acc.at[pl.ds(sid * RPT, RPT)],
                    out_hbm.at[cid, pl.ds(sid * RPT, RPT)])


_sc_gcn = pl.kernel(
    _sc_gcn_body,
    out_type=jax.ShapeDtypeStruct((NC, NPAD, C), jnp.float32),
    mesh=_MESH,
    scratch_types=[
        pltpu.VMEM((GCN_CH,), jnp.int32),
        pltpu.VMEM((GCN_CH,), jnp.int32),
        pltpu.VMEM((GCN_CH, C), jnp.float32),
        pltpu.VMEM((16, C), jnp.float32),
        pltpu.VMEM_SHARED((NPAD, C), jnp.float32),
        pltpu.SemaphoreType.DMA,
    ],
)


def _cgconv(h, src, dst, et, Wp, Wq, g, be):
    p = _mm(h, Wp)
    q = _mm(h, Wq)
    agg, deg_raw = _sc_cg(p, q, et, dst, src)
    return _cg_epilogue(agg[0, :N], agg[1, :N], h, g, be), deg_raw


def _gcnconv(h, src, dst, dis_n, W, b):
    xs = _mm(h, W, rs=dis_n)
    agg = _sc_gcn(xs, dst, src)
    return _gcn_epilogue(agg[0, :N], agg[1, :N], xs, dis_n, b)


def kernel(x, edge_index, edge_attr, goal_feat, batch, Wf1, bf1, Ws1, bs1, g1, be1, Wf2, bf2, Ws2, bs2, g2, be2, W3, b3, W4, b4, Wd1, bd1, Wd2, bd2):
    src, dst = edge_index[0], edge_index[1]

    Wp1 = jnp.concatenate([Wf1[:C], Ws1[:C]], axis=1)
    Wq1 = jnp.concatenate([Wf1[C:2 * C], Ws1[C:2 * C]], axis=1)
    Wet1 = jnp.concatenate([Wf1[2 * C:], Ws1[2 * C:]], axis=1)
    bet1 = jnp.concatenate([bf1, bs1])
    Wp2 = jnp.concatenate([Wf2[:C], Ws2[:C]], axis=1)
    Wq2 = jnp.concatenate([Wf2[C:2 * C], Ws2[C:2 * C]], axis=1)
    Wet2 = jnp.concatenate([Wf2[2 * C:], Ws2[2 * C:]], axis=1)
    bet2 = jnp.concatenate([bf2, bs2])

    et1 = _mm(edge_attr, Wet1, bet1)
    et2 = _mm(edge_attr, Wet2, bet2)

    h, deg_raw = _cgconv(x, src, dst, et1, Wp1, Wq1, g1, be1)
    h, _ = _cgconv(h, src, dst, et2, Wp2, Wq2, g2, be2)

    dis_n = _deg_finish(deg_raw)[:N, None]

    h = _gcnconv(h, src, dst, dis_n, W3, b3)
    h = _gcnconv(h, src, dst, dis_n, W4, b4)

    t = _mm2(h, goal_feat, Wd1[:C], Wd1[C:], bd1, act="relu")
    Wd2p = jnp.concatenate([Wd2, jnp.zeros((C, 127), jnp.float32)], axis=1)
    bd2p = jnp.concatenate([bd2, jnp.zeros((127,), jnp.float32)])
    pred = _mm(t, Wd2p, bd2p)[:, :1]
    return (pred, h)
